# baseline (speedup 1.0000x reference)
import jax
import jax.numpy as jnp
from jax.experimental import pallas as pl
from jax.experimental.pallas import tpu as pltpu

_N = 8192
_TA = 1024
_TB = 2048
_NI = _N // _TA


def _loss_kernel(a_ref, a8_ref, bT8_ref, bT_ref, p3T_ref, out_ref, colmin_ref):
    c = pl.program_id(0)
    i = pl.program_id(1)

    a = a_ref[0]
    a8 = a8_ref[0]
    bT8 = bT8_ref[...]
    bT = bT_ref[...]

    a2 = jnp.sum(a * a, axis=1, keepdims=True)

    w = jnp.where(c == 0, 0.5, 1.0)

    @pl.when(jnp.logical_and(c == 0, i == 0))
    def _init_out():
        diff = p3T_ref[...] - bT
        out_ref[0, 0] = jnp.sum(diff * diff) / (_N * 3)

    @pl.when(i == 0)
    def _init_colmin():
        colmin_ref[...] = jnp.full((1, _N), jnp.inf, jnp.float32)

    row_min = jnp.full((_TA, 1), jnp.inf, jnp.float32)
    for j in range(_N // _TB):
        sl = pl.ds(j * _TB, _TB)
        m = jax.lax.dot_general(
            a8, bT8_ref[:, sl],
            dimension_numbers=(((1,), (0,)), ((), ())),
            preferred_element_type=jnp.float32,
        )
        bTj = bT[:, j * _TB:(j + 1) * _TB]
        b2 = jnp.sum(bTj * bTj, axis=0, keepdims=True)
        v = (m + b2) + a2
        row_min = jnp.minimum(
            row_min, jnp.min(v, axis=1, keepdims=True)
        )
        colmin_ref[:, sl] = jnp.minimum(
            colmin_ref[:, sl], jnp.min(v, axis=0, keepdims=True)
        )
    row_min = jnp.maximum(row_min, 0.0)

    total = w * jnp.sum(jnp.sqrt(row_min)) / _N
    out_ref[0, 0] = out_ref[0, 0] + total

    @pl.when(i == _NI - 1)
    def _finish_col():
        col_final = jnp.maximum(colmin_ref[...], 0.0)
        col_sum = jnp.sum(jnp.sqrt(col_final)) / _N
        out_ref[0, 0] = out_ref[0, 0] + w * col_sum


def kernel(pc1, pc2):
    a01 = pc1[:2]
    a8 = jnp.zeros((2, _N, 8), jnp.bfloat16)
    a8 = a8.at[:, :, :3].set((-2.0 * a01).astype(jnp.bfloat16))
    bT = pc2.T
    bT8 = jnp.zeros((8, _N), jnp.bfloat16)
    bT8 = bT8.at[:3, :].set(bT.astype(jnp.bfloat16))
    p3T = pc1[3].T

    out = pl.pallas_call(
        _loss_kernel,
        grid=(2, _NI),
        in_specs=[
            pl.BlockSpec((1, _TA, 3), lambda c, i: (c, i, 0)),
            pl.BlockSpec((1, _TA, 8), lambda c, i: (c, i, 0)),
            pl.BlockSpec((8, _N), lambda c, i: (0, 0)),
            pl.BlockSpec((3, _N), lambda c, i: (0, 0)),
            pl.BlockSpec((3, _N), lambda c, i: (0, 0)),
        ],
        out_specs=pl.BlockSpec(memory_space=pltpu.SMEM),
        out_shape=jax.ShapeDtypeStruct((1, 1), jnp.float32),
        scratch_shapes=[pltpu.VMEM((1, _N), jnp.float32)],
        compiler_params=pltpu.CompilerParams(
            dimension_semantics=("arbitrary", "arbitrary"),
        ),
    )(a01, a8, bT8, bT, p3T)
    return out[0, 0]


# speedup vs baseline: 2.1666x; 2.1666x over previous
import jax
import jax.numpy as jnp
from jax.experimental import pallas as pl
from jax.experimental.pallas import tpu as pltpu

_N = 8192
_TA = 1024
_TB = 4096
_NI = _N // _TA
_K = 16


def _loss_kernel(a16_ref, bT16_ref, bT_ref, p3T_ref, out_ref, colmin_ref):
    c = pl.program_id(0)
    i = pl.program_id(1)

    a16 = a16_ref[0]
    bT = bT_ref[...]

    w = jnp.where(c == 0, 0.5, 1.0)

    @pl.when(jnp.logical_and(c == 0, i == 0))
    def _init_out():
        diff = p3T_ref[...] - bT
        out_ref[0, 0] = jnp.sum(diff * diff) / (_N * 3)

    @pl.when(i == 0)
    def _init_colmin():
        colmin_ref[...] = jnp.full((1, _N), jnp.inf, jnp.float32)

    v = jax.lax.dot_general(
        a16, bT16_ref[...],
        dimension_numbers=(((1,), (0,)), ((), ())),
        preferred_element_type=jnp.float32,
    )
    row_min = jnp.maximum(jnp.min(v, axis=1, keepdims=True), 0.0)
    colmin_ref[...] = jnp.minimum(
        colmin_ref[...], jnp.min(v, axis=0, keepdims=True)
    )

    total = w * jnp.sum(jnp.sqrt(row_min)) / _N
    out_ref[0, 0] = out_ref[0, 0] + total

    @pl.when(i == _NI - 1)
    def _finish_col():
        col_final = jnp.maximum(colmin_ref[...], 0.0)
        col_sum = jnp.sum(jnp.sqrt(col_final)) / _N
        out_ref[0, 0] = out_ref[0, 0] + w * col_sum


def _bf16_triple_split(x):
    h1 = x.astype(jnp.bfloat16).astype(jnp.float32)
    r1 = x - h1
    h2 = r1.astype(jnp.bfloat16).astype(jnp.float32)
    h3 = (r1 - h2).astype(jnp.bfloat16)
    return h1.astype(jnp.bfloat16), h2.astype(jnp.bfloat16), h3


def kernel(pc1, pc2):
    a01 = pc1[:2]
    bT = pc2.T
    p3T = pc1[3].T

    a2 = jnp.sum(a01 * a01, axis=2)
    b2 = jnp.sum(pc2 * pc2, axis=1)
    ah1, ah2, ah3 = _bf16_triple_split(a2)
    bh1, bh2, bh3 = _bf16_triple_split(b2)

    ones_a = jnp.ones((2, _N), jnp.bfloat16)
    a16 = jnp.zeros((2, _N, _K), jnp.bfloat16)
    a16 = a16.at[:, :, :3].set((-2.0 * a01).astype(jnp.bfloat16))
    a16 = a16.at[:, :, 3].set(ah1)
    a16 = a16.at[:, :, 4].set(ah2)
    a16 = a16.at[:, :, 5].set(ah3)
    a16 = a16.at[:, :, 6].set(ones_a)
    a16 = a16.at[:, :, 7].set(ones_a)
    a16 = a16.at[:, :, 8].set(ones_a)

    ones_b = jnp.ones((_N,), jnp.bfloat16)
    bT16 = jnp.zeros((_K, _N), jnp.bfloat16)
    bT16 = bT16.at[:3, :].set(bT.astype(jnp.bfloat16))
    bT16 = bT16.at[3, :].set(ones_b)
    bT16 = bT16.at[4, :].set(ones_b)
    bT16 = bT16.at[5, :].set(ones_b)
    bT16 = bT16.at[6, :].set(bh1)
    bT16 = bT16.at[7, :].set(bh2)
    bT16 = bT16.at[8, :].set(bh3)

    out = pl.pallas_call(
        _loss_kernel,
        grid=(2, _NI),
        in_specs=[
            pl.BlockSpec((1, _TA, _K), lambda c, i: (c, i, 0)),
            pl.BlockSpec((_K, _N), lambda c, i: (0, 0)),
            pl.BlockSpec((3, _N), lambda c, i: (0, 0)),
            pl.BlockSpec((3, _N), lambda c, i: (0, 0)),
        ],
        out_specs=pl.BlockSpec(memory_space=pltpu.SMEM),
        out_shape=jax.ShapeDtypeStruct((1, 1), jnp.float32),
        scratch_shapes=[pltpu.VMEM((1, _N), jnp.float32)],
        compiler_params=pltpu.CompilerParams(
            dimension_semantics=("arbitrary", "arbitrary"),
        ),
    )(a16, bT16, bT, p3T)
    return out[0, 0]
